# baseline (speedup 1.0000x reference)
import functools

import jax
import jax.numpy as jnp
from jax import lax
from jax.experimental import pallas as pl
from jax.experimental.pallas import tpu as pltpu
from jax.experimental.pallas import tpu_sc as plsc

HIDDEN = 128
N = 320000
NUM_SEG = 10000
SEG_PAD = 10240

ROWS_A = 4096
HALF_BLOCKS_1 = 28
HALF_ROWS_1 = HALF_BLOCKS_1 * ROWS_A
HALF_BLOCKS_2 = -(-(N - HALF_ROWS_1) // ROWS_A)


def _logits_body(x_ref, w1_ref, b1_ref, w2_ref, e_ref):
    x = x_ref[...]
    h = lax.dot_general(x, w1_ref[...], (((1,), (1,)), ((), ())),
                        preferred_element_type=jnp.float32)
    h = jnp.maximum(h + b1_ref[...], 0.0)
    logit_t = lax.dot_general(w2_ref[...], h, (((1,), (1,)), ((), ())),
                              preferred_element_type=jnp.float32)
    e_ref[...] = jnp.exp(jnp.reshape(logit_t, (ROWS_A,)))


def _phase_a(x, w1, b1, w2, start_block, num_blocks):
    return pl.pallas_call(
        _logits_body,
        grid=(num_blocks,),
        in_specs=[
            pl.BlockSpec((ROWS_A, HIDDEN), lambda i: (i + start_block, 0)),
            pl.BlockSpec((HIDDEN // 2, HIDDEN), lambda i: (0, 0)),
            pl.BlockSpec((1, HIDDEN // 2), lambda i: (0, 0)),
            pl.BlockSpec((1, HIDDEN // 2), lambda i: (0, 0)),
        ],
        out_specs=pl.BlockSpec((ROWS_A,), lambda i: (i,)),
        out_shape=jax.ShapeDtypeStruct((num_blocks * ROWS_A,), jnp.float32),
    )(x, w1, b1.reshape(1, -1), w2)


CHUNK = 128
NW = 32
ZERO_ROWS = SEG_PAD // 16
ZCHUNK = 128


def _make_sc_body(ch0, nchunks, row_off):
    ch_per_w = nchunks // NW
    extra = nchunks - NW * ch_per_w
    return functools.partial(_sc_scatter_body, ch0=ch0, ch_per_w=ch_per_w,
                             extra=extra, row_off=row_off)


def _sc_scatter_body(x_hbm, e_hbm, seg_hbm, p_hbm, z_hbm,
                     rows_v, e_v, idx_v, acc_p, acc_z,
                     sem_l0, sem_l1, sem_s0, sem_s1,
                     *, ch0, ch_per_w, extra, row_off):
    cid = lax.axis_index("c")
    sid = lax.axis_index("s")
    wid = cid * 16 + sid
    sem_l = (sem_l0, sem_l1)
    sem_s = (sem_s0, sem_s1)

    def zero_row(i, _):
        for j in range(HIDDEN // 16):
            rows_v[0, i, pl.ds(j * 16, 16)] = jnp.zeros((16,), jnp.float32)
        return 0
    lax.fori_loop(0, CHUNK, zero_row, 0)
    for j in range(CHUNK // 16):
        e_v[0, pl.ds(j * 16, 16)] = jnp.zeros((16,), jnp.float32)

    for j in range(ZERO_ROWS // ZCHUNK):
        pltpu.sync_copy(
            rows_v.at[0],
            acc_p.at[pl.ds(sid * ZERO_ROWS + j * ZCHUNK, ZCHUNK)])
        pltpu.sync_copy(
            e_v.at[0],
            acc_z.at[pl.ds(sid * ZERO_ROWS + j * ZCHUNK, ZCHUNK)])
    plsc.subcore_barrier()

    def issue_loads(c, b):
        r0 = c * CHUNK
        pltpu.async_copy(seg_hbm.at[pl.ds(r0, CHUNK)], idx_v.at[b], sem_l[b])
        pltpu.async_copy(e_hbm.at[pl.ds(r0 - row_off, CHUNK)], e_v.at[b],
                         sem_l[b])
        pltpu.async_copy(x_hbm.at[pl.ds(r0, CHUNK)], rows_v.at[b], sem_l[b])

    def wait_loads(c, b):
        r0 = c * CHUNK
        pltpu.make_async_copy(
            seg_hbm.at[pl.ds(r0, CHUNK)], idx_v.at[b], sem_l[b]).wait()
        pltpu.make_async_copy(
            e_hbm.at[pl.ds(r0 - row_off, CHUNK)], e_v.at[b], sem_l[b]).wait()
        pltpu.make_async_copy(
            x_hbm.at[pl.ds(r0, CHUNK)], rows_v.at[b], sem_l[b]).wait()

    def multiply(b):
        @plsc.parallel_loop(0, CHUNK // 16, 1)
        def _(g):
            e16 = e_v[b, pl.ds(g * 16, 16)]
            for t in range(16):
                s = e16[t]
                r = g * 16 + t
                for i in range(HIDDEN // 16):
                    rows_v[b, r, pl.ds(i * 16, 16)] = (
                        rows_v[b, r, pl.ds(i * 16, 16)] * s)

    def issue_scatter(b):
        pltpu.async_copy(rows_v.at[b], acc_p.at[idx_v.at[b]], sem_s[b],
                         add=True)
        pltpu.async_copy(e_v.at[b], acc_z.at[idx_v.at[b]], sem_s[b],
                         add=True)

    def wait_scatter(b):
        pltpu.make_async_copy(
            rows_v.at[b], acc_p.at[idx_v.at[b]], sem_s[b]).wait()
        pltpu.make_async_copy(
            e_v.at[b], acc_z.at[idx_v.at[b]], sem_s[b]).wait()

    base = ch0 + wid * ch_per_w
    issue_loads(base, 0)
    wait_loads(base, 0)
    issue_loads(base + 1, 1)
    multiply(0)
    issue_scatter(0)

    def pair(k, _):
        c0 = base + 1 + 2 * k
        wait_loads(c0, 1)
        wait_scatter(0)
        issue_loads(c0 + 1, 0)
        multiply(1)
        issue_scatter(1)
        c1 = c0 + 1
        wait_loads(c1, 0)
        wait_scatter(1)
        issue_loads(c1 + 1, 1)
        multiply(0)
        issue_scatter(0)
        return 0
    lax.fori_loop(0, (ch_per_w - 2) // 2, pair, 0)

    wait_loads(base + ch_per_w - 1, 1)
    wait_scatter(0)
    multiply(1)
    issue_scatter(1)
    wait_scatter(1)

    if extra:
        @pl.when(wid < extra)
        def _leftover():
            c = ch0 + NW * ch_per_w + wid
            r0 = c * CHUNK
            pltpu.sync_copy(seg_hbm.at[pl.ds(r0, CHUNK)], idx_v.at[0])
            pltpu.sync_copy(e_hbm.at[pl.ds(r0 - row_off, CHUNK)], e_v.at[0])
            pltpu.sync_copy(x_hbm.at[pl.ds(r0, CHUNK)], rows_v.at[0])
            multiply(0)
            pltpu.sync_copy(rows_v.at[0], acc_p.at[idx_v.at[0]], add=True)
            pltpu.sync_copy(e_v.at[0], acc_z.at[idx_v.at[0]], add=True)

    plsc.subcore_barrier()

    pltpu.sync_copy(acc_p.at[pl.ds(sid * ZERO_ROWS, ZERO_ROWS)],
                    p_hbm.at[cid, pl.ds(sid * ZERO_ROWS, ZERO_ROWS)])
    pltpu.sync_copy(acc_z.at[pl.ds(sid * ZERO_ROWS, ZERO_ROWS)],
                    z_hbm.at[cid, pl.ds(sid * ZERO_ROWS, ZERO_ROWS)])


def _phase_b(x, e_flat, seg, ch0, nchunks, row_off):
    mesh = plsc.VectorSubcoreMesh(core_axis_name="c", subcore_axis_name="s")
    f = pl.kernel(
        _make_sc_body(ch0, nchunks, row_off),
        out_type=[
            jax.ShapeDtypeStruct((2, SEG_PAD, HIDDEN), jnp.float32),
            jax.ShapeDtypeStruct((2, SEG_PAD), jnp.float32),
        ],
        mesh=mesh,
        scratch_types=[
            pltpu.VMEM((2, CHUNK, HIDDEN), jnp.float32),
            pltpu.VMEM((2, CHUNK), jnp.float32),
            pltpu.VMEM((2, CHUNK), jnp.int32),
            pltpu.VMEM_SHARED((SEG_PAD, HIDDEN), jnp.float32),
            pltpu.VMEM_SHARED((SEG_PAD,), jnp.float32),
            pltpu.SemaphoreType.DMA,
            pltpu.SemaphoreType.DMA,
            pltpu.SemaphoreType.DMA,
            pltpu.SemaphoreType.DMA,
        ],
    )
    return f(x, e_flat, seg)


ROWS_C = 2048


def _head_body(p1_ref, z1_ref, p2_ref, z2_ref, w3_ref, b3_ref, w4_ref,
               b4_ref, w5_ref, b5_ref, o_ref):
    p = (p1_ref[0] + p1_ref[1]) + (p2_ref[0] + p2_ref[1])
    z = (z1_ref[0] + z1_ref[1]) + (z2_ref[0] + z2_ref[1])
    mol = jnp.where(z > 0.0, p / jnp.where(z > 0.0, z, 1.0), 0.0)
    g = lax.dot_general(mol, w3_ref[...], (((1,), (1,)), ((), ())),
                        preferred_element_type=jnp.float32)
    g = jnp.maximum(g + b3_ref[...], 0.0)
    g = lax.dot_general(g, w4_ref[...], (((1,), (1,)), ((), ())),
                        preferred_element_type=jnp.float32)
    g = jnp.maximum(g + b4_ref[...], 0.0)
    o = lax.dot_general(g, w5_ref[...], (((1,), (1,)), ((), ())),
                        preferred_element_type=jnp.float32)
    o_ref[...] = o + b5_ref[0]


def _phase_c(p1, z1, p2, z2, w3, b3, w4, b4, w5, b5):
    grid = SEG_PAD // ROWS_C
    return pl.pallas_call(
        _head_body,
        grid=(grid,),
        in_specs=[
            pl.BlockSpec((2, ROWS_C, HIDDEN), lambda i: (0, i, 0)),
            pl.BlockSpec((2, ROWS_C, 1), lambda i: (0, i, 0)),
            pl.BlockSpec((2, ROWS_C, HIDDEN), lambda i: (0, i, 0)),
            pl.BlockSpec((2, ROWS_C, 1), lambda i: (0, i, 0)),
            pl.BlockSpec((HIDDEN, HIDDEN), lambda i: (0, 0)),
            pl.BlockSpec((1, HIDDEN), lambda i: (0, 0)),
            pl.BlockSpec((HIDDEN // 2, HIDDEN), lambda i: (0, 0)),
            pl.BlockSpec((1, HIDDEN // 2), lambda i: (0, 0)),
            pl.BlockSpec((8, HIDDEN // 2), lambda i: (0, 0)),
            pl.BlockSpec(memory_space=pltpu.SMEM),
        ],
        out_specs=pl.BlockSpec((ROWS_C, 8), lambda i: (i, 0)),
        out_shape=jax.ShapeDtypeStruct((SEG_PAD, 8), jnp.float32),
    )(p1, z1.reshape(2, SEG_PAD, 1), p2, z2.reshape(2, SEG_PAD, 1),
      w3, b3.reshape(1, -1), w4, b4.reshape(1, -1),
      jnp.pad(w5, ((0, 7), (0, 0))), b5)


def kernel(orbital_embeddings, batch, W1, b1, W2, b2, W3, b3, W4, b4, W5, b5):
    seg = batch.astype(jnp.int32)
    x = orbital_embeddings
    nch1 = HALF_ROWS_1 // CHUNK
    nch2 = N // CHUNK - nch1
    e1 = _phase_a(x, W1, b1, W2, 0, HALF_BLOCKS_1)
    e2 = _phase_a(x, W1, b1, W2, HALF_BLOCKS_1, HALF_BLOCKS_2)
    p1, z1 = _phase_b(x, e1, seg, 0, nch1, 0)
    p2, z2 = _phase_b(x, e2, seg, nch1, nch2, HALF_ROWS_1)
    out = _phase_c(p1, z1, p2, z2, W3, b3, W4, b4, W5, b5)
    return out[:NUM_SEG, :1]


# speedup vs baseline: 1.0866x; 1.0866x over previous
import functools

import jax
import jax.numpy as jnp
from jax import lax
from jax.experimental import pallas as pl
from jax.experimental.pallas import tpu as pltpu
from jax.experimental.pallas import tpu_sc as plsc

HIDDEN = 128
N = 320000
NUM_SEG = 10000
SEG_PAD = 10240

ROWS_A = 4096
HALF_BLOCKS_1 = 40
HALF_ROWS_1 = HALF_BLOCKS_1 * ROWS_A
HALF_BLOCKS_2 = -(-(N - HALF_ROWS_1) // ROWS_A)


def _logits_body(x_ref, w1_ref, b1_ref, w2_ref, e_ref):
    x = x_ref[...]
    h = lax.dot_general(x, w1_ref[...], (((1,), (1,)), ((), ())),
                        preferred_element_type=jnp.float32)
    h = jnp.maximum(h + b1_ref[...], 0.0)
    logit_t = lax.dot_general(w2_ref[...], h, (((1,), (1,)), ((), ())),
                              preferred_element_type=jnp.float32)
    e_ref[...] = jnp.exp(jnp.reshape(logit_t, (ROWS_A,)))


def _phase_a(x, w1, b1, w2, start_block, num_blocks):
    return pl.pallas_call(
        _logits_body,
        grid=(num_blocks,),
        in_specs=[
            pl.BlockSpec((ROWS_A, HIDDEN), lambda i: (i + start_block, 0)),
            pl.BlockSpec((HIDDEN // 2, HIDDEN), lambda i: (0, 0)),
            pl.BlockSpec((1, HIDDEN // 2), lambda i: (0, 0)),
            pl.BlockSpec((1, HIDDEN // 2), lambda i: (0, 0)),
        ],
        out_specs=pl.BlockSpec((ROWS_A,), lambda i: (i,)),
        out_shape=jax.ShapeDtypeStruct((num_blocks * ROWS_A,), jnp.float32),
    )(x, w1, b1.reshape(1, -1), w2)


CHUNK = 128
NW = 32
ZERO_ROWS = SEG_PAD // 16
ZCHUNK = 128


def _make_sc_body(ch0, nchunks, row_off):
    ch_per_w = nchunks // NW
    extra = nchunks - NW * ch_per_w
    return functools.partial(_sc_scatter_body, ch0=ch0, ch_per_w=ch_per_w,
                             extra=extra, row_off=row_off)


def _sc_scatter_body(x_hbm, e_hbm, seg_hbm, p_hbm, z_hbm,
                     rows_v, e_v, idx_v, acc_p, acc_z,
                     sem_l0, sem_l1, sem_s0, sem_s1,
                     *, ch0, ch_per_w, extra, row_off):
    cid = lax.axis_index("c")
    sid = lax.axis_index("s")
    wid = cid * 16 + sid
    sem_l = (sem_l0, sem_l1)
    sem_s = (sem_s0, sem_s1)

    def zero_row(i, _):
        for j in range(HIDDEN // 16):
            rows_v[0, i, pl.ds(j * 16, 16)] = jnp.zeros((16,), jnp.float32)
        return 0
    lax.fori_loop(0, CHUNK, zero_row, 0)
    for j in range(CHUNK // 16):
        e_v[0, pl.ds(j * 16, 16)] = jnp.zeros((16,), jnp.float32)

    for j in range(ZERO_ROWS // ZCHUNK):
        pltpu.sync_copy(
            rows_v.at[0],
            acc_p.at[pl.ds(sid * ZERO_ROWS + j * ZCHUNK, ZCHUNK)])
        pltpu.sync_copy(
            e_v.at[0],
            acc_z.at[pl.ds(sid * ZERO_ROWS + j * ZCHUNK, ZCHUNK)])
    plsc.subcore_barrier()

    def issue_loads(c, b):
        r0 = c * CHUNK
        pltpu.async_copy(seg_hbm.at[pl.ds(r0, CHUNK)], idx_v.at[b], sem_l[b])
        pltpu.async_copy(e_hbm.at[pl.ds(r0 - row_off, CHUNK)], e_v.at[b],
                         sem_l[b])
        pltpu.async_copy(x_hbm.at[pl.ds(r0, CHUNK)], rows_v.at[b], sem_l[b])

    def wait_loads(c, b):
        r0 = c * CHUNK
        pltpu.make_async_copy(
            seg_hbm.at[pl.ds(r0, CHUNK)], idx_v.at[b], sem_l[b]).wait()
        pltpu.make_async_copy(
            e_hbm.at[pl.ds(r0 - row_off, CHUNK)], e_v.at[b], sem_l[b]).wait()
        pltpu.make_async_copy(
            x_hbm.at[pl.ds(r0, CHUNK)], rows_v.at[b], sem_l[b]).wait()

    def multiply(b):
        @plsc.parallel_loop(0, CHUNK // 16, 1)
        def _(g):
            e16 = e_v[b, pl.ds(g * 16, 16)]
            for t in range(16):
                s = e16[t]
                r = g * 16 + t
                for i in range(HIDDEN // 16):
                    rows_v[b, r, pl.ds(i * 16, 16)] = (
                        rows_v[b, r, pl.ds(i * 16, 16)] * s)

    def issue_scatter(b):
        pltpu.async_copy(rows_v.at[b], acc_p.at[idx_v.at[b]], sem_s[b],
                         add=True)
        pltpu.async_copy(e_v.at[b], acc_z.at[idx_v.at[b]], sem_s[b],
                         add=True)

    def wait_scatter(b):
        pltpu.make_async_copy(
            rows_v.at[b], acc_p.at[idx_v.at[b]], sem_s[b]).wait()
        pltpu.make_async_copy(
            e_v.at[b], acc_z.at[idx_v.at[b]], sem_s[b]).wait()

    base = ch0 + wid * ch_per_w
    issue_loads(base, 0)
    wait_loads(base, 0)
    issue_loads(base + 1, 1)
    multiply(0)
    issue_scatter(0)

    def pair(k, _):
        c0 = base + 1 + 2 * k
        wait_loads(c0, 1)
        wait_scatter(0)
        issue_loads(c0 + 1, 0)
        multiply(1)
        issue_scatter(1)
        c1 = c0 + 1
        wait_loads(c1, 0)
        wait_scatter(1)
        issue_loads(c1 + 1, 1)
        multiply(0)
        issue_scatter(0)
        return 0
    lax.fori_loop(0, (ch_per_w - 2) // 2, pair, 0)

    wait_loads(base + ch_per_w - 1, 1)
    wait_scatter(0)
    multiply(1)
    issue_scatter(1)
    wait_scatter(1)

    if extra:
        @pl.when(wid < extra)
        def _leftover():
            c = ch0 + NW * ch_per_w + wid
            r0 = c * CHUNK
            pltpu.sync_copy(seg_hbm.at[pl.ds(r0, CHUNK)], idx_v.at[0])
            pltpu.sync_copy(e_hbm.at[pl.ds(r0 - row_off, CHUNK)], e_v.at[0])
            pltpu.sync_copy(x_hbm.at[pl.ds(r0, CHUNK)], rows_v.at[0])
            multiply(0)
            pltpu.sync_copy(rows_v.at[0], acc_p.at[idx_v.at[0]], add=True)
            pltpu.sync_copy(e_v.at[0], acc_z.at[idx_v.at[0]], add=True)

    plsc.subcore_barrier()

    pltpu.sync_copy(acc_p.at[pl.ds(sid * ZERO_ROWS, ZERO_ROWS)],
                    p_hbm.at[cid, pl.ds(sid * ZERO_ROWS, ZERO_ROWS)])
    pltpu.sync_copy(acc_z.at[pl.ds(sid * ZERO_ROWS, ZERO_ROWS)],
                    z_hbm.at[cid, pl.ds(sid * ZERO_ROWS, ZERO_ROWS)])


def _phase_b(x, e_flat, seg, ch0, nchunks, row_off):
    mesh = plsc.VectorSubcoreMesh(core_axis_name="c", subcore_axis_name="s")
    f = pl.kernel(
        _make_sc_body(ch0, nchunks, row_off),
        out_type=[
            jax.ShapeDtypeStruct((2, SEG_PAD, HIDDEN), jnp.float32),
            jax.ShapeDtypeStruct((2, SEG_PAD), jnp.float32),
        ],
        mesh=mesh,
        scratch_types=[
            pltpu.VMEM((2, CHUNK, HIDDEN), jnp.float32),
            pltpu.VMEM((2, CHUNK), jnp.float32),
            pltpu.VMEM((2, CHUNK), jnp.int32),
            pltpu.VMEM_SHARED((SEG_PAD, HIDDEN), jnp.float32),
            pltpu.VMEM_SHARED((SEG_PAD,), jnp.float32),
            pltpu.SemaphoreType.DMA,
            pltpu.SemaphoreType.DMA,
            pltpu.SemaphoreType.DMA,
            pltpu.SemaphoreType.DMA,
        ],
    )
    return f(x, e_flat, seg)


ROWS_C = 2048


def _head_body(p1_ref, z1_ref, p2_ref, z2_ref, w3_ref, b3_ref, w4_ref,
               b4_ref, w5_ref, b5_ref, o_ref):
    p = (p1_ref[0] + p1_ref[1]) + (p2_ref[0] + p2_ref[1])
    zc = z1_ref[...] + z2_ref[...]
    zsum = lax.dot_general(zc, jnp.ones((2, 8), jnp.float32),
                           (((0,), (0,)), ((), ())),
                           preferred_element_type=jnp.float32)
    z = zsum[:, 0:1]
    mol = jnp.where(z > 0.0, p / jnp.where(z > 0.0, z, 1.0), 0.0)
    g = lax.dot_general(mol, w3_ref[...], (((1,), (1,)), ((), ())),
                        preferred_element_type=jnp.float32)
    g = jnp.maximum(g + b3_ref[...], 0.0)
    g = lax.dot_general(g, w4_ref[...], (((1,), (1,)), ((), ())),
                        preferred_element_type=jnp.float32)
    g = jnp.maximum(g + b4_ref[...], 0.0)
    o = lax.dot_general(g, w5_ref[...], (((1,), (1,)), ((), ())),
                        preferred_element_type=jnp.float32)
    o_ref[...] = o + b5_ref[0]


def _phase_c(p1, z1, p2, z2, w3, b3, w4, b4, w5, b5):
    grid = SEG_PAD // ROWS_C
    return pl.pallas_call(
        _head_body,
        grid=(grid,),
        in_specs=[
            pl.BlockSpec((2, ROWS_C, HIDDEN), lambda i: (0, i, 0)),
            pl.BlockSpec((2, ROWS_C), lambda i: (0, i)),
            pl.BlockSpec((2, ROWS_C, HIDDEN), lambda i: (0, i, 0)),
            pl.BlockSpec((2, ROWS_C), lambda i: (0, i)),
            pl.BlockSpec((HIDDEN, HIDDEN), lambda i: (0, 0)),
            pl.BlockSpec((1, HIDDEN), lambda i: (0, 0)),
            pl.BlockSpec((HIDDEN // 2, HIDDEN), lambda i: (0, 0)),
            pl.BlockSpec((1, HIDDEN // 2), lambda i: (0, 0)),
            pl.BlockSpec((8, HIDDEN // 2), lambda i: (0, 0)),
            pl.BlockSpec(memory_space=pltpu.SMEM),
        ],
        out_specs=pl.BlockSpec((ROWS_C, 8), lambda i: (i, 0)),
        out_shape=jax.ShapeDtypeStruct((SEG_PAD, 8), jnp.float32),
    )(p1, z1, p2, z2,
      w3, b3.reshape(1, -1), w4, b4.reshape(1, -1),
      jnp.pad(w5, ((0, 7), (0, 0))), b5)


def kernel(orbital_embeddings, batch, W1, b1, W2, b2, W3, b3, W4, b4, W5, b5):
    seg = batch.astype(jnp.int32)
    x = orbital_embeddings
    nch1 = HALF_ROWS_1 // CHUNK
    nch2 = N // CHUNK - nch1
    e1 = _phase_a(x, W1, b1, W2, 0, HALF_BLOCKS_1)
    e2 = _phase_a(x, W1, b1, W2, HALF_BLOCKS_1, HALF_BLOCKS_2)
    p1, z1 = _phase_b(x, e1, seg, 0, nch1, 0)
    p2, z2 = _phase_b(x, e2, seg, nch1, nch2, HALF_ROWS_1)
    out = _phase_c(p1, z1, p2, z2, W3, b3, W4, b4, W5, b5)
    return out[:NUM_SEG, :1]


# speedup vs baseline: 1.0973x; 1.0098x over previous
import functools

import jax
import jax.numpy as jnp
from jax import lax
from jax.experimental import pallas as pl
from jax.experimental.pallas import tpu as pltpu
from jax.experimental.pallas import tpu_sc as plsc

HIDDEN = 128
N = 320000
NUM_SEG = 10000
SEG_PAD = 10240

ROWS_A = 4096
HALF_BLOCKS_1 = 40
HALF_ROWS_1 = HALF_BLOCKS_1 * ROWS_A
HALF_BLOCKS_2 = -(-(N - HALF_ROWS_1) // ROWS_A)


def _logits_body(x_ref, w1_ref, b1_ref, w2_ref, e_ref):
    x = x_ref[...]
    h = lax.dot_general(x, w1_ref[...], (((1,), (1,)), ((), ())),
                        preferred_element_type=jnp.float32)
    h = jnp.maximum(h + b1_ref[...], 0.0)
    logit_t = lax.dot_general(w2_ref[...], h, (((1,), (1,)), ((), ())),
                              preferred_element_type=jnp.float32)
    e_ref[...] = jnp.exp(jnp.reshape(logit_t, (ROWS_A,)))


def _phase_a(x, w1, b1, w2, start_block, num_blocks):
    return pl.pallas_call(
        _logits_body,
        grid=(num_blocks,),
        in_specs=[
            pl.BlockSpec((ROWS_A, HIDDEN), lambda i: (i + start_block, 0)),
            pl.BlockSpec((HIDDEN // 2, HIDDEN), lambda i: (0, 0)),
            pl.BlockSpec((1, HIDDEN // 2), lambda i: (0, 0)),
            pl.BlockSpec((1, HIDDEN // 2), lambda i: (0, 0)),
        ],
        out_specs=pl.BlockSpec((ROWS_A,), lambda i: (i,)),
        out_shape=jax.ShapeDtypeStruct((num_blocks * ROWS_A,), jnp.float32),
    )(x, w1, b1.reshape(1, -1), w2)


CHUNK = 128
NW = 32
ZERO_ROWS = SEG_PAD // 16
ZCHUNK = 128


def _make_sc_body(ch0, nchunks, row_off):
    ch_per_w = nchunks // NW
    extra = nchunks - NW * ch_per_w
    return functools.partial(_sc_scatter_body, ch0=ch0, ch_per_w=ch_per_w,
                             extra=extra, row_off=row_off)


def _sc_scatter_body(x_hbm, e_hbm, seg_hbm, p_hbm, z_hbm,
                     rows_v, e_v, idx_v, acc_p, acc_z,
                     sem_l0, sem_l1, sem_s0, sem_s1,
                     *, ch0, ch_per_w, extra, row_off):
    cid = lax.axis_index("c")
    sid = lax.axis_index("s")
    wid = cid * 16 + sid
    sem_l = (sem_l0, sem_l1)
    sem_s = (sem_s0, sem_s1)

    def zero_row(i, _):
        for j in range(HIDDEN // 16):
            rows_v[0, i, pl.ds(j * 16, 16)] = jnp.zeros((16,), jnp.float32)
        return 0
    lax.fori_loop(0, CHUNK, zero_row, 0)
    for j in range(CHUNK // 16):
        e_v[0, pl.ds(j * 16, 16)] = jnp.zeros((16,), jnp.float32)

    for j in range(ZERO_ROWS // ZCHUNK):
        pltpu.sync_copy(
            rows_v.at[0],
            acc_p.at[pl.ds(sid * ZERO_ROWS + j * ZCHUNK, ZCHUNK)])
        pltpu.sync_copy(
            e_v.at[0],
            acc_z.at[pl.ds(sid * ZERO_ROWS + j * ZCHUNK, ZCHUNK)])
    plsc.subcore_barrier()

    def issue_loads(c, b):
        r0 = c * CHUNK
        pltpu.async_copy(seg_hbm.at[pl.ds(r0, CHUNK)], idx_v.at[b], sem_l[b])
        pltpu.async_copy(e_hbm.at[pl.ds(r0 - row_off, CHUNK)], e_v.at[b],
                         sem_l[b])
        pltpu.async_copy(x_hbm.at[pl.ds(r0, CHUNK)], rows_v.at[b], sem_l[b])

    def wait_loads(c, b):
        r0 = c * CHUNK
        pltpu.make_async_copy(
            seg_hbm.at[pl.ds(r0, CHUNK)], idx_v.at[b], sem_l[b]).wait()
        pltpu.make_async_copy(
            e_hbm.at[pl.ds(r0 - row_off, CHUNK)], e_v.at[b], sem_l[b]).wait()
        pltpu.make_async_copy(
            x_hbm.at[pl.ds(r0, CHUNK)], rows_v.at[b], sem_l[b]).wait()

    def multiply(b):
        @plsc.parallel_loop(0, CHUNK // 16, 1, unroll=2)
        def _(g):
            e16 = e_v[b, pl.ds(g * 16, 16)]
            for t in range(16):
                s = e16[t]
                r = g * 16 + t
                for i in range(HIDDEN // 16):
                    rows_v[b, r, pl.ds(i * 16, 16)] = (
                        rows_v[b, r, pl.ds(i * 16, 16)] * s)

    def issue_scatter(b):
        pltpu.async_copy(e_v.at[b], acc_z.at[idx_v.at[b]], sem_s[b],
                         add=True)
        pltpu.async_copy(rows_v.at[b], acc_p.at[idx_v.at[b]], sem_s[b],
                         add=True)

    def wait_scatter(b):
        pltpu.make_async_copy(
            rows_v.at[b], acc_p.at[idx_v.at[b]], sem_s[b]).wait()
        pltpu.make_async_copy(
            e_v.at[b], acc_z.at[idx_v.at[b]], sem_s[b]).wait()

    base = ch0 + wid * ch_per_w
    issue_loads(base, 0)
    wait_loads(base, 0)
    issue_loads(base + 1, 1)
    multiply(0)
    issue_scatter(0)

    def pair(k, _):
        c0 = base + 1 + 2 * k
        wait_loads(c0, 1)
        wait_scatter(0)
        issue_loads(c0 + 1, 0)
        multiply(1)
        issue_scatter(1)
        c1 = c0 + 1
        wait_loads(c1, 0)
        wait_scatter(1)
        issue_loads(c1 + 1, 1)
        multiply(0)
        issue_scatter(0)
        return 0
    lax.fori_loop(0, (ch_per_w - 2) // 2, pair, 0)

    wait_loads(base + ch_per_w - 1, 1)
    wait_scatter(0)
    multiply(1)
    issue_scatter(1)
    wait_scatter(1)

    if extra:
        @pl.when(wid < extra)
        def _leftover():
            c = ch0 + NW * ch_per_w + wid
            r0 = c * CHUNK
            pltpu.sync_copy(seg_hbm.at[pl.ds(r0, CHUNK)], idx_v.at[0])
            pltpu.sync_copy(e_hbm.at[pl.ds(r0 - row_off, CHUNK)], e_v.at[0])
            pltpu.sync_copy(x_hbm.at[pl.ds(r0, CHUNK)], rows_v.at[0])
            multiply(0)
            pltpu.sync_copy(rows_v.at[0], acc_p.at[idx_v.at[0]], add=True)
            pltpu.sync_copy(e_v.at[0], acc_z.at[idx_v.at[0]], add=True)

    plsc.subcore_barrier()

    pltpu.sync_copy(acc_p.at[pl.ds(sid * ZERO_ROWS, ZERO_ROWS)],
                    p_hbm.at[cid, pl.ds(sid * ZERO_ROWS, ZERO_ROWS)])
    pltpu.sync_copy(acc_z.at[pl.ds(sid * ZERO_ROWS, ZERO_ROWS)],
                    z_hbm.at[cid, pl.ds(sid * ZERO_ROWS, ZERO_ROWS)])


def _phase_b(x, e_flat, seg, ch0, nchunks, row_off):
    mesh = plsc.VectorSubcoreMesh(core_axis_name="c", subcore_axis_name="s")
    f = pl.kernel(
        _make_sc_body(ch0, nchunks, row_off),
        out_type=[
            jax.ShapeDtypeStruct((2, SEG_PAD, HIDDEN), jnp.float32),
            jax.ShapeDtypeStruct((2, SEG_PAD), jnp.float32),
        ],
        mesh=mesh,
        scratch_types=[
            pltpu.VMEM((2, CHUNK, HIDDEN), jnp.float32),
            pltpu.VMEM((2, CHUNK), jnp.float32),
            pltpu.VMEM((2, CHUNK), jnp.int32),
            pltpu.VMEM_SHARED((SEG_PAD, HIDDEN), jnp.float32),
            pltpu.VMEM_SHARED((SEG_PAD,), jnp.float32),
            pltpu.SemaphoreType.DMA,
            pltpu.SemaphoreType.DMA,
            pltpu.SemaphoreType.DMA,
            pltpu.SemaphoreType.DMA,
        ],
    )
    return f(x, e_flat, seg)


ROWS_C = 2048


def _head_body(p1_ref, z1_ref, p2_ref, z2_ref, w3_ref, b3_ref, w4_ref,
               b4_ref, w5_ref, b5_ref, o_ref):
    p = (p1_ref[0] + p1_ref[1]) + (p2_ref[0] + p2_ref[1])
    zc = z1_ref[...] + z2_ref[...]
    zsum = lax.dot_general(zc, jnp.ones((2, 8), jnp.float32),
                           (((0,), (0,)), ((), ())),
                           preferred_element_type=jnp.float32)
    z = zsum[:, 0:1]
    mol = jnp.where(z > 0.0, p / jnp.where(z > 0.0, z, 1.0), 0.0)
    g = lax.dot_general(mol, w3_ref[...], (((1,), (1,)), ((), ())),
                        preferred_element_type=jnp.float32)
    g = jnp.maximum(g + b3_ref[...], 0.0)
    g = lax.dot_general(g, w4_ref[...], (((1,), (1,)), ((), ())),
                        preferred_element_type=jnp.float32)
    g = jnp.maximum(g + b4_ref[...], 0.0)
    o = lax.dot_general(g, w5_ref[...], (((1,), (1,)), ((), ())),
                        preferred_element_type=jnp.float32)
    o_ref[...] = o + b5_ref[0]


def _phase_c(p1, z1, p2, z2, w3, b3, w4, b4, w5, b5):
    grid = SEG_PAD // ROWS_C
    return pl.pallas_call(
        _head_body,
        grid=(grid,),
        in_specs=[
            pl.BlockSpec((2, ROWS_C, HIDDEN), lambda i: (0, i, 0)),
            pl.BlockSpec((2, ROWS_C), lambda i: (0, i)),
            pl.BlockSpec((2, ROWS_C, HIDDEN), lambda i: (0, i, 0)),
            pl.BlockSpec((2, ROWS_C), lambda i: (0, i)),
            pl.BlockSpec((HIDDEN, HIDDEN), lambda i: (0, 0)),
            pl.BlockSpec((1, HIDDEN), lambda i: (0, 0)),
            pl.BlockSpec((HIDDEN // 2, HIDDEN), lambda i: (0, 0)),
            pl.BlockSpec((1, HIDDEN // 2), lambda i: (0, 0)),
            pl.BlockSpec((8, HIDDEN // 2), lambda i: (0, 0)),
            pl.BlockSpec(memory_space=pltpu.SMEM),
        ],
        out_specs=pl.BlockSpec((ROWS_C, 8), lambda i: (i, 0)),
        out_shape=jax.ShapeDtypeStruct((SEG_PAD, 8), jnp.float32),
    )(p1, z1, p2, z2,
      w3, b3.reshape(1, -1), w4, b4.reshape(1, -1),
      jnp.pad(w5, ((0, 7), (0, 0))), b5)


def kernel(orbital_embeddings, batch, W1, b1, W2, b2, W3, b3, W4, b4, W5, b5):
    seg = batch.astype(jnp.int32)
    x = orbital_embeddings
    nch1 = HALF_ROWS_1 // CHUNK
    nch2 = N // CHUNK - nch1
    e1 = _phase_a(x, W1, b1, W2, 0, HALF_BLOCKS_1)
    e2 = _phase_a(x, W1, b1, W2, HALF_BLOCKS_1, HALF_BLOCKS_2)
    p1, z1 = _phase_b(x, e1, seg, 0, nch1, 0)
    p2, z2 = _phase_b(x, e2, seg, nch1, nch2, HALF_ROWS_1)
    out = _phase_c(p1, z1, p2, z2, W3, b3, W4, b4, W5, b5)
    return out[:NUM_SEG, :1]
